# Initial kernel scaffold; baseline (speedup 1.0000x reference)
import jax
import jax.numpy as jnp
from jax.experimental import pallas as pl


def kernel(coords, pe):
    raise NotImplementedError("write your pallas kernel here")


# speedup vs baseline: 1.1720x; 1.1720x over previous
import functools

import jax
import jax.numpy as jnp
from jax import lax
from jax.experimental import pallas as pl
from jax.experimental.pallas import tpu as pltpu
from jax.experimental.pallas import tpu_sc as plsc

NC = 2
NS = 16
L = 16
NW = NC * NS

N = 16 * 128 * 50
D = 256
MAX_LEN = 100
B_PER_W = N // NW
CHUNK = 128
NCHUNK = B_PER_W // CHUNK
QVECS = B_PER_W // L

_mesh = plsc.VectorSubcoreMesh(
    core_axis_name="c", subcore_axis_name="s", num_cores=NC, num_subcores=NS
)


def _pair_body(pe_ref, out_ref):
    i = pl.program_id(0)
    row = pe_ref[pl.ds(i, 1), :]
    out_ref[...] = (row[:, None, :] + pe_ref[...][None, :, :])


_pair_table = pl.pallas_call(
    _pair_body,
    grid=(MAX_LEN,),
    in_specs=[pl.BlockSpec((MAX_LEN, D), lambda i: (0, 0))],
    out_specs=pl.BlockSpec((1, MAX_LEN, D), lambda i: (i, 0, 0)),
    out_shape=jax.ShapeDtypeStruct((MAX_LEN, MAX_LEN, D), jnp.float32),
)


@functools.partial(
    pl.kernel,
    out_type=jax.ShapeDtypeStruct((N, D), jnp.float32),
    mesh=_mesh,
    scratch_types=[
        pltpu.VMEM((B_PER_W,), jnp.float32),
        pltpu.VMEM((B_PER_W,), jnp.float32),
        pltpu.VMEM((NCHUNK, CHUNK), jnp.int32),
        pltpu.VMEM((CHUNK, D), jnp.float32),
        pltpu.SemaphoreType.DMA,
    ],
)
def _sc_embed(x_hbm, y_hbm, pair_hbm, out_hbm, xv, yv, cidx, rows, sem):
    wid = lax.axis_index("s") * NC + lax.axis_index("c")
    base = wid * B_PER_W
    pltpu.sync_copy(x_hbm.at[pl.ds(base, B_PER_W)], xv)
    pltpu.sync_copy(y_hbm.at[pl.ds(base, B_PER_W)], yv)

    def _quant(v):
        norm = (v + 50.0) / 100.0
        return jnp.clip((norm * float(MAX_LEN - 1)).astype(jnp.int32), 0, MAX_LEN - 1)

    def qbody(i, carry):
        c = i // (CHUNK // L)
        k = i % (CHUNK // L)
        qx = _quant(xv[pl.ds(i * L, L)])
        qy = _quant(yv[pl.ds(i * L, L)])
        cidx[c, pl.ds(k * L, L)] = qx * MAX_LEN + qy
        return carry

    lax.fori_loop(0, QVECS, qbody, 0)

    def gbody(c, carry):
        pltpu.async_copy(pair_hbm.at[cidx.at[c]], rows, sem).wait()
        pltpu.sync_copy(rows, out_hbm.at[pl.ds(base + c * CHUNK, CHUNK)])
        return carry

    lax.fori_loop(0, NCHUNK, gbody, 0)


def kernel(coords, pe):
    lead = coords.shape[:-1]
    x = coords[..., 0].reshape(-1)
    y = coords[..., 1].reshape(-1)
    pair = _pair_table(pe).reshape(MAX_LEN * MAX_LEN, D)
    out = _sc_embed(x, y, pair)
    return out.reshape(lead + (pe.shape[1],))
